# baseline (speedup 1.0000x reference)
import functools

import jax
import jax.numpy as jnp
from jax import lax
from jax.experimental import pallas as pl
from jax.experimental.pallas import tpu as pltpu
from jax.experimental.pallas import tpu_sc as plsc

SEQLEN = 65536
TOKENSIZE = 256
WINDOW = 4096

N = WINDOW * TOKENSIZE
FLAT = SEQLEN * TOKENSIZE
NC, NS = 2, 16
NW = NC * NS
CHUNK = N // NW


@functools.partial(
    pl.kernel,
    mesh=plsc.VectorSubcoreMesh(core_axis_name="c", subcore_axis_name="s"),
    out_type=jax.ShapeDtypeStruct((2 * N,), jnp.float32),
    scratch_types=[
        pltpu.VMEM((CHUNK,), jnp.int32),
        pltpu.VMEM((CHUNK,), jnp.float32),
        pltpu.VMEM((CHUNK,), jnp.float32),
        pltpu.SemaphoreType.DMA,
        pltpu.SemaphoreType.DMA,
    ],
)
def _sc_permute(table_hbm, idx_hbm, out_hbm, idx_v, val_v, cpy_v, sem, sem2):
    wid = lax.axis_index("s") * NC + lax.axis_index("c")
    base = wid * CHUNK
    cp_in = pltpu.make_async_copy(table_hbm.at[pl.ds(base, CHUNK)], cpy_v, sem2)
    cp_in.start()
    pltpu.sync_copy(idx_hbm.at[pl.ds(base, CHUNK)], idx_v)
    gather = pltpu.make_async_copy(table_hbm.at[idx_v], val_v, sem)
    gather.start()
    cp_in.wait()
    pltpu.sync_copy(cpy_v, out_hbm.at[pl.ds(base, CHUNK)])
    gather.wait()
    pltpu.sync_copy(val_v, out_hbm.at[pl.ds(N + base, CHUNK)])


_ROWS_PER_BLK = 512


def _tc_copy_body(mdx_ref, p_ref, mdx_out_ref, p_out_ref):
    mdx_out_ref[...] = mdx_ref[...]
    p_out_ref[...] = p_ref[...]


def _tc_window_copies(mdx, p):
    blk = pl.BlockSpec((_ROWS_PER_BLK, TOKENSIZE), lambda i: (i, 0))
    return pl.pallas_call(
        _tc_copy_body,
        grid=(WINDOW // _ROWS_PER_BLK,),
        in_specs=[blk, blk],
        out_specs=[blk, blk],
        out_shape=[
            jax.ShapeDtypeStruct((WINDOW, TOKENSIZE), jnp.float32),
            jax.ShapeDtypeStruct((WINDOW, TOKENSIZE), jnp.float32),
        ],
    )(mdx, p)


def kernel(ddx, mdx, p, perm):
    table = ddx.reshape(FLAT)
    idx = perm.reshape(FLAT)
    out = _sc_permute(table, idx)
    ddx_out = out.reshape(2, WINDOW, TOKENSIZE)
    mdx_out, p_out = _tc_window_copies(mdx, p)
    return (ddx_out, mdx_out, p_out)


# speedup vs baseline: 1.3947x; 1.3947x over previous
import functools

import jax
import jax.numpy as jnp
from jax import lax
from jax.experimental import pallas as pl
from jax.experimental.pallas import tpu as pltpu
from jax.experimental.pallas import tpu_sc as plsc

SEQLEN = 65536
TOKENSIZE = 256
WINDOW = 4096

N = WINDOW * TOKENSIZE
FLAT = SEQLEN * TOKENSIZE
NC, NS = 2, 16
NW = NC * NS
CHUNK = N // NW


@functools.partial(
    pl.kernel,
    mesh=plsc.VectorSubcoreMesh(core_axis_name="c", subcore_axis_name="s"),
    out_type=jax.ShapeDtypeStruct((2 * N,), jnp.float32),
    scratch_types=[
        pltpu.VMEM((CHUNK,), jnp.int32),
        pltpu.VMEM((CHUNK,), jnp.float32),
        pltpu.VMEM((CHUNK,), jnp.float32),
        pltpu.SemaphoreType.DMA,
        pltpu.SemaphoreType.DMA,
    ],
)
def _sc_permute(table_hbm, idx_hbm, out_hbm, idx_v, val_v, cpy_v, sem, sem2):
    wid = lax.axis_index("s") * NC + lax.axis_index("c")
    base = wid * CHUNK
    cp_in = pltpu.make_async_copy(table_hbm.at[pl.ds(base, CHUNK)], cpy_v, sem2)
    cp_in.start()
    pltpu.sync_copy(idx_hbm.at[pl.ds(base, CHUNK)], idx_v)
    gather = pltpu.make_async_copy(table_hbm.at[idx_v], val_v, sem)
    gather.start()
    cp_in.wait()
    pltpu.sync_copy(cpy_v, out_hbm.at[pl.ds(base, CHUNK)])
    gather.wait()
    pltpu.sync_copy(val_v, out_hbm.at[pl.ds(N + base, CHUNK)])


_ROWS_PER_BLK = 512


def _tc_copy_body(mdx_ref, p_ref, mdx_out_ref, p_out_ref):
    mdx_out_ref[...] = mdx_ref[...]
    p_out_ref[...] = p_ref[...]


def _tc_window_copies(mdx, p):
    blk = pl.BlockSpec((_ROWS_PER_BLK, TOKENSIZE), lambda i: (i, 0))
    return pl.pallas_call(
        _tc_copy_body,
        grid=(WINDOW // _ROWS_PER_BLK,),
        in_specs=[blk, blk],
        out_specs=[blk, blk],
        out_shape=[
            jax.ShapeDtypeStruct((WINDOW, TOKENSIZE), jnp.float32),
            jax.ShapeDtypeStruct((WINDOW, TOKENSIZE), jnp.float32),
        ],
    )(mdx, p)


def kernel(ddx, mdx, p, perm):
    table = ddx.reshape(FLAT)
    idx = jax.lax.slice(perm, (0, 0), (WINDOW, TOKENSIZE)).reshape(N)
    out = _sc_permute(table, idx)
    ddx_out = out.reshape(2, WINDOW, TOKENSIZE)
    mdx_out, p_out = _tc_window_copies(mdx, p)
    return (ddx_out, mdx_out, p_out)
